# baseline (speedup 1.0000x reference)
import functools

import jax
import jax.numpy as jnp
from jax import lax
from jax.experimental import pallas as pl
from jax.experimental.pallas import tpu as pltpu
from jax.experimental.pallas import tpu_sc as plsc


def _lane_shuffle(x, idx):
    dnums = lax.GatherDimensionNumbers(
        offset_dims=(), collapsed_slice_dims=(0,), start_index_map=(0,))
    return lax.gather(x, idx[:, None], dnums, slice_sizes=(1,),
                      mode=lax.GatherScatterMode.PROMISE_IN_BOUNDS)


def _widen(u):
    lo = lax.bitcast_convert_type(u << 16, jnp.float32)
    hi = lax.bitcast_convert_type(u & jnp.int32(-65536), jnp.float32)
    return lo, hi


def _narrow(a, b):
    ua = lax.shift_right_logical(
        lax.bitcast_convert_type(a, jnp.int32) + jnp.int32(0x8000), 16)
    ub = (lax.bitcast_convert_type(b, jnp.int32) + jnp.int32(0x8000)) \
        & jnp.int32(-65536)
    return ub | (ua & jnp.int32(0xFFFF))


N_NODES_ = 10000
N_EDGES_ = 320000
D_ = 128

NC_ = 2
NS_ = 16
C_ = 64
NBUF_ = 4
NCHUNKS_ = N_EDGES_ // C_
CH_A_ = 209
CH_B_ = NCHUNKS_ // NS_ - CH_A_ - 1
_FAST_TOTAL_ = NS_ * CH_A_ + 8
CH_B_ = (NCHUNKS_ - _FAST_TOTAL_) // NS_
T_SRC_A_, T_DST_A_ = 4, 3
T_SRC_B_, T_DST_B_ = 5, 5
ROWS_T_ = N_NODES_ // NS_
RPK_ = 25


def _sc_kernel(hs, hd, ei, out, hpk_s, hpk_d,
               idx_s, idx_d, rows_s, rows_d, score, pin, pout, sh_s, sh_d,
               sem_i, sem_s, sem_d, sem_o):
    c = lax.axis_index("c")
    s = lax.axis_index("s")
    n = jnp.where(c == 0, CH_A_ + (s < 8), CH_B_).astype(jnp.int32)
    base = jnp.where(c == 0, s * CH_A_ + jnp.minimum(s, 8),
                     _FAST_TOTAL_ + s * CH_B_).astype(jnp.int32)
    t_src = jnp.where(c == 0, T_SRC_A_, T_SRC_B_).astype(jnp.int32)
    t_dst = jnp.where(c == 0, T_DST_A_, T_DST_B_).astype(jnp.int32)

    row0 = s * ROWS_T_

    def _pack(src_tab, dst_sh, dst_hbm):
        @pl.loop(0, ROWS_T_ // RPK_)
        def _blk(bk):
            r0 = row0 + bk * RPK_
            pltpu.sync_copy(src_tab.at[pl.ds(r0, RPK_)], pin)

            @pl.loop(0, RPK_)
            def _row(i):
                for q in range(4):
                    a = pin[i, 32 * q:32 * q + 16]
                    b = pin[i, 32 * q + 16:32 * q + 32]
                    pout[i, 16 * q:16 * (q + 1)] = _narrow(a, b)

            pltpu.sync_copy(pout, dst_sh.at[pl.ds(r0, RPK_)])
            pltpu.sync_copy(pout, dst_hbm.at[c, pl.ds(r0, RPK_)])

    _pack(hs, sh_s, hpk_s)
    _pack(hd, sh_d, hpk_d)
    plsc.subcore_barrier()

    lane = lax.iota(jnp.int32, 16)
    perms = [(lane + sh) & 15 for sh in (8, 4, 2, 1)]

    def _issue_idx(k):
        slot = k % NBUF_
        off = (base + k) * C_
        pltpu.async_copy(ei.at[0, pl.ds(off, C_)], idx_s.at[slot],
                         sem_i.at[slot])
        pltpu.async_copy(ei.at[1, pl.ds(off, C_)], idx_d.at[slot],
                         sem_i.at[slot])

    def _wait_idx(k):
        slot = k % NBUF_
        pltpu.make_async_copy(ei.at[0, pl.ds((base + k) * C_, C_)],
                              idx_s.at[slot], sem_i.at[slot]).wait()
        pltpu.make_async_copy(ei.at[1, pl.ds((base + k) * C_, C_)],
                              idx_d.at[slot], sem_i.at[slot]).wait()

    def _routes(k):
        sp_src = ((k * t_src) % 8) < t_src
        sp_dst = ((k * t_dst) % 8) < t_dst
        return sp_src, sp_dst

    def _issue_gather(k):
        slot = k % NBUF_
        sp_src, sp_dst = _routes(k)

        @pl.when(sp_src)
        def _():
            pltpu.async_copy(sh_s.at[idx_s.at[slot]], rows_s.at[slot],
                             sem_s.at[slot])

        @pl.when(jnp.logical_not(sp_src))
        def _():
            pltpu.async_copy(hpk_s.at[c].at[idx_s.at[slot]], rows_s.at[slot],
                             sem_s.at[slot])

        @pl.when(sp_dst)
        def _():
            pltpu.async_copy(sh_d.at[idx_d.at[slot]], rows_d.at[slot],
                             sem_d.at[slot])

        @pl.when(jnp.logical_not(sp_dst))
        def _():
            pltpu.async_copy(hpk_d.at[c].at[idx_d.at[slot]], rows_d.at[slot],
                             sem_d.at[slot])

    def _wait_gather(k):
        slot = k % NBUF_
        sp_src, sp_dst = _routes(k)

        @pl.when(sp_src)
        def _():
            pltpu.make_async_copy(sh_s.at[idx_s.at[slot]], rows_s.at[slot],
                                  sem_s.at[slot]).wait()

        @pl.when(jnp.logical_not(sp_src))
        def _():
            pltpu.make_async_copy(hpk_s.at[c].at[idx_s.at[slot]],
                                  rows_s.at[slot], sem_s.at[slot]).wait()

        @pl.when(sp_dst)
        def _():
            pltpu.make_async_copy(sh_d.at[idx_d.at[slot]], rows_d.at[slot],
                                  sem_d.at[slot]).wait()

        @pl.when(jnp.logical_not(sp_dst))
        def _():
            pltpu.make_async_copy(hpk_d.at[c].at[idx_d.at[slot]],
                                  rows_d.at[slot], sem_d.at[slot]).wait()

    def _issue_out(k):
        slot = k % NBUF_
        pltpu.async_copy(score.at[slot], out.at[pl.ds((base + k) * C_, C_)],
                         sem_o.at[slot])

    def _wait_out(k):
        slot = k % NBUF_
        pltpu.make_async_copy(score.at[slot],
                              out.at[pl.ds((base + k) * C_, C_)],
                              sem_o.at[slot]).wait()

    for j in range(NBUF_):
        _issue_idx(j)
    for j in range(NBUF_ - 1):
        _wait_idx(j)
        _issue_gather(j)

    @pl.loop(0, n)
    def _chunk(k):
        slot = k % NBUF_
        _wait_gather(k)

        @pl.when(k >= NBUF_)
        def _drain_out():
            _wait_out(k - NBUF_)

        @pl.loop(0, C_ // 16)
        def _group(g):
            @pl.loop(0, 16, init_carry=jnp.zeros((16,), jnp.float32))
            def _edge(u, sc_vec):
                e = g * 16 + u
                p = []
                for f in range(4):
                    a0, a1 = _widen(rows_s[slot, e, 16 * f:16 * (f + 1)])
                    b0, b1 = _widen(rows_d[slot, e, 16 * f:16 * (f + 1)])
                    p.append(a0 * b0)
                    p.append(a1 * b1)
                acc = ((p[0] + p[1]) + (p[2] + p[3])) + \
                      ((p[4] + p[5]) + (p[6] + p[7]))
                for pm in perms:
                    acc = acc + _lane_shuffle(acc, pm)
                return jnp.where(lane == u, acc, sc_vec)
            score[slot, pl.ds(g * 16, 16)] = _edge

        _issue_out(k)

        @pl.when(k + NBUF_ < n)
        def _refill_idx():
            _issue_idx(k + NBUF_)

        @pl.when(k + NBUF_ - 1 < n)
        def _refill_gather():
            _wait_idx(k + NBUF_ - 1)
            _issue_gather(k + NBUF_ - 1)

    for j in range(NBUF_):
        _wait_out(n - NBUF_ + j)


@jax.jit
def kernel(h_src, h_dst, edge_index):
    ei = edge_index.astype(jnp.int32)

    mesh = plsc.VectorSubcoreMesh(core_axis_name="c", subcore_axis_name="s")
    sck = functools.partial(
        pl.kernel,
        out_type=(
            jax.ShapeDtypeStruct((N_EDGES_,), jnp.float32),
            jax.ShapeDtypeStruct((NC_, N_NODES_, D_ // 2), jnp.int32),
            jax.ShapeDtypeStruct((NC_, N_NODES_, D_ // 2), jnp.int32),
        ),
        mesh=mesh,
        compiler_params=pltpu.CompilerParams(use_tc_tiling_on_sc=False),
        scratch_types=[
            pltpu.VMEM((NBUF_, C_), jnp.int32),
            pltpu.VMEM((NBUF_, C_), jnp.int32),
            pltpu.VMEM((NBUF_, C_, D_ // 2), jnp.int32),
            pltpu.VMEM((NBUF_, C_, D_ // 2), jnp.int32),
            pltpu.VMEM((NBUF_, C_), jnp.float32),
            pltpu.VMEM((RPK_, D_), jnp.float32),
            pltpu.VMEM((RPK_, D_ // 2), jnp.int32),
            pltpu.VMEM_SHARED((N_NODES_, D_ // 2), jnp.int32),
            pltpu.VMEM_SHARED((N_NODES_, D_ // 2), jnp.int32),
            pltpu.SemaphoreType.DMA((NBUF_,)),
            pltpu.SemaphoreType.DMA((NBUF_,)),
            pltpu.SemaphoreType.DMA((NBUF_,)),
            pltpu.SemaphoreType.DMA((NBUF_,)),
        ],
    )(_sc_kernel)
    out, _, _ = sck(h_src, h_dst, ei)
    return out.reshape(N_EDGES_, 1)


# speedup vs baseline: 1.0513x; 1.0513x over previous
import functools

import jax
import jax.numpy as jnp
from jax import lax
from jax.experimental import pallas as pl
from jax.experimental.pallas import tpu as pltpu
from jax.experimental.pallas import tpu_sc as plsc


def _lane_shuffle(x, idx):
    dnums = lax.GatherDimensionNumbers(
        offset_dims=(), collapsed_slice_dims=(0,), start_index_map=(0,))
    return lax.gather(x, idx[:, None], dnums, slice_sizes=(1,),
                      mode=lax.GatherScatterMode.PROMISE_IN_BOUNDS)


def _widen(u):
    lo = lax.bitcast_convert_type(u << 16, jnp.float32)
    hi = lax.bitcast_convert_type(u & jnp.int32(-65536), jnp.float32)
    return lo, hi


def _narrow(a, b):
    ua = lax.shift_right_logical(
        lax.bitcast_convert_type(a, jnp.int32) + jnp.int32(0x8000), 16)
    ub = (lax.bitcast_convert_type(b, jnp.int32) + jnp.int32(0x8000)) \
        & jnp.int32(-65536)
    return ub | (ua & jnp.int32(0xFFFF))


N_NODES_ = 10000
N_EDGES_ = 320000
D_ = 128

NC_ = 2
NS_ = 16
C_ = 64
NBUF_ = 4
NCHUNKS_ = N_EDGES_ // C_
CH_A_ = 190
CH_B_ = NCHUNKS_ // NS_ - CH_A_ - 1
_FAST_TOTAL_ = NS_ * CH_A_ + 8
CH_B_ = (NCHUNKS_ - _FAST_TOTAL_) // NS_
T_SRC_A_, T_DST_A_ = 4, 3
T_SRC_B_, T_DST_B_ = 6, 6
ROWS_T_ = N_NODES_ // NS_
RPK_ = 25


def _sc_kernel(hs, hd, ei, out, hpk_s, hpk_d,
               idx_s, idx_d, rows_s, rows_d, score, pin, pout, sh_s, sh_d,
               sem_i, sem_s, sem_d, sem_o):
    c = lax.axis_index("c")
    s = lax.axis_index("s")
    n = jnp.where(c == 0, CH_A_ + (s < 8), CH_B_).astype(jnp.int32)
    base = jnp.where(c == 0, s * CH_A_ + jnp.minimum(s, 8),
                     _FAST_TOTAL_ + s * CH_B_).astype(jnp.int32)
    t_src = jnp.where(c == 0, T_SRC_A_, T_SRC_B_).astype(jnp.int32)
    t_dst = jnp.where(c == 0, T_DST_A_, T_DST_B_).astype(jnp.int32)

    row0 = s * ROWS_T_

    def _pack(src_tab, dst_sh, dst_hbm):
        @pl.loop(0, ROWS_T_ // RPK_)
        def _blk(bk):
            r0 = row0 + bk * RPK_
            pltpu.sync_copy(src_tab.at[pl.ds(r0, RPK_)], pin)

            @pl.loop(0, RPK_)
            def _row(i):
                for q in range(4):
                    a = pin[i, 32 * q:32 * q + 16]
                    b = pin[i, 32 * q + 16:32 * q + 32]
                    pout[i, 16 * q:16 * (q + 1)] = _narrow(a, b)

            pltpu.sync_copy(pout, dst_sh.at[pl.ds(r0, RPK_)])
            pltpu.sync_copy(pout, dst_hbm.at[c, pl.ds(r0, RPK_)])

    _pack(hs, sh_s, hpk_s)
    _pack(hd, sh_d, hpk_d)
    plsc.subcore_barrier()

    lane = lax.iota(jnp.int32, 16)
    perms = [(lane + sh) & 15 for sh in (8, 4, 2, 1)]

    def _issue_idx(k):
        slot = k % NBUF_
        off = (base + k) * C_
        pltpu.async_copy(ei.at[0, pl.ds(off, C_)], idx_s.at[slot],
                         sem_i.at[slot])
        pltpu.async_copy(ei.at[1, pl.ds(off, C_)], idx_d.at[slot],
                         sem_i.at[slot])

    def _wait_idx(k):
        slot = k % NBUF_
        pltpu.make_async_copy(ei.at[0, pl.ds((base + k) * C_, C_)],
                              idx_s.at[slot], sem_i.at[slot]).wait()
        pltpu.make_async_copy(ei.at[1, pl.ds((base + k) * C_, C_)],
                              idx_d.at[slot], sem_i.at[slot]).wait()

    def _routes(k):
        sp_src = ((k * t_src) % 8) < t_src
        sp_dst = ((k * t_dst) % 8) < t_dst
        return sp_src, sp_dst

    def _issue_gather(k):
        slot = k % NBUF_
        sp_src, sp_dst = _routes(k)

        @pl.when(sp_src)
        def _():
            pltpu.async_copy(sh_s.at[idx_s.at[slot]], rows_s.at[slot],
                             sem_s.at[slot])

        @pl.when(jnp.logical_not(sp_src))
        def _():
            pltpu.async_copy(hpk_s.at[c].at[idx_s.at[slot]], rows_s.at[slot],
                             sem_s.at[slot])

        @pl.when(sp_dst)
        def _():
            pltpu.async_copy(sh_d.at[idx_d.at[slot]], rows_d.at[slot],
                             sem_d.at[slot])

        @pl.when(jnp.logical_not(sp_dst))
        def _():
            pltpu.async_copy(hpk_d.at[c].at[idx_d.at[slot]], rows_d.at[slot],
                             sem_d.at[slot])

    def _wait_gather(k):
        slot = k % NBUF_
        sp_src, sp_dst = _routes(k)

        @pl.when(sp_src)
        def _():
            pltpu.make_async_copy(sh_s.at[idx_s.at[slot]], rows_s.at[slot],
                                  sem_s.at[slot]).wait()

        @pl.when(jnp.logical_not(sp_src))
        def _():
            pltpu.make_async_copy(hpk_s.at[c].at[idx_s.at[slot]],
                                  rows_s.at[slot], sem_s.at[slot]).wait()

        @pl.when(sp_dst)
        def _():
            pltpu.make_async_copy(sh_d.at[idx_d.at[slot]], rows_d.at[slot],
                                  sem_d.at[slot]).wait()

        @pl.when(jnp.logical_not(sp_dst))
        def _():
            pltpu.make_async_copy(hpk_d.at[c].at[idx_d.at[slot]],
                                  rows_d.at[slot], sem_d.at[slot]).wait()

    def _issue_out(k):
        slot = k % NBUF_
        pltpu.async_copy(score.at[slot], out.at[pl.ds((base + k) * C_, C_)],
                         sem_o.at[slot])

    def _wait_out(k):
        slot = k % NBUF_
        pltpu.make_async_copy(score.at[slot],
                              out.at[pl.ds((base + k) * C_, C_)],
                              sem_o.at[slot]).wait()

    for j in range(NBUF_):
        _issue_idx(j)
    for j in range(NBUF_ - 1):
        _wait_idx(j)
        _issue_gather(j)

    @pl.loop(0, n)
    def _chunk(k):
        slot = k % NBUF_
        _wait_gather(k)

        @pl.when(k >= NBUF_)
        def _drain_out():
            _wait_out(k - NBUF_)

        @pl.loop(0, C_ // 16)
        def _group(g):
            @pl.loop(0, 16, init_carry=jnp.zeros((16,), jnp.float32))
            def _edge(u, sc_vec):
                e = g * 16 + u
                p = []
                for f in range(4):
                    a0, a1 = _widen(rows_s[slot, e, 16 * f:16 * (f + 1)])
                    b0, b1 = _widen(rows_d[slot, e, 16 * f:16 * (f + 1)])
                    p.append(a0 * b0)
                    p.append(a1 * b1)
                acc = ((p[0] + p[1]) + (p[2] + p[3])) + \
                      ((p[4] + p[5]) + (p[6] + p[7]))
                for pm in perms:
                    acc = acc + _lane_shuffle(acc, pm)
                return jnp.where(lane == u, acc, sc_vec)
            score[slot, pl.ds(g * 16, 16)] = _edge

        _issue_out(k)

        @pl.when(k + NBUF_ < n)
        def _refill_idx():
            _issue_idx(k + NBUF_)

        @pl.when(k + NBUF_ - 1 < n)
        def _refill_gather():
            _wait_idx(k + NBUF_ - 1)
            _issue_gather(k + NBUF_ - 1)

    for j in range(NBUF_):
        _wait_out(n - NBUF_ + j)


@jax.jit
def kernel(h_src, h_dst, edge_index):
    ei = edge_index.astype(jnp.int32)

    mesh = plsc.VectorSubcoreMesh(core_axis_name="c", subcore_axis_name="s")
    sck = functools.partial(
        pl.kernel,
        out_type=(
            jax.ShapeDtypeStruct((N_EDGES_,), jnp.float32),
            jax.ShapeDtypeStruct((NC_, N_NODES_, D_ // 2), jnp.int32),
            jax.ShapeDtypeStruct((NC_, N_NODES_, D_ // 2), jnp.int32),
        ),
        mesh=mesh,
        compiler_params=pltpu.CompilerParams(use_tc_tiling_on_sc=False),
        scratch_types=[
            pltpu.VMEM((NBUF_, C_), jnp.int32),
            pltpu.VMEM((NBUF_, C_), jnp.int32),
            pltpu.VMEM((NBUF_, C_, D_ // 2), jnp.int32),
            pltpu.VMEM((NBUF_, C_, D_ // 2), jnp.int32),
            pltpu.VMEM((NBUF_, C_), jnp.float32),
            pltpu.VMEM((RPK_, D_), jnp.float32),
            pltpu.VMEM((RPK_, D_ // 2), jnp.int32),
            pltpu.VMEM_SHARED((N_NODES_, D_ // 2), jnp.int32),
            pltpu.VMEM_SHARED((N_NODES_, D_ // 2), jnp.int32),
            pltpu.SemaphoreType.DMA((NBUF_,)),
            pltpu.SemaphoreType.DMA((NBUF_,)),
            pltpu.SemaphoreType.DMA((NBUF_,)),
            pltpu.SemaphoreType.DMA((NBUF_,)),
        ],
    )(_sc_kernel)
    out, _, _ = sck(h_src, h_dst, ei)
    return out.reshape(N_EDGES_, 1)


# speedup vs baseline: 1.0578x; 1.0062x over previous
import functools

import jax
import jax.numpy as jnp
from jax import lax
from jax.experimental import pallas as pl
from jax.experimental.pallas import tpu as pltpu
from jax.experimental.pallas import tpu_sc as plsc


def _lane_shuffle(x, idx):
    dnums = lax.GatherDimensionNumbers(
        offset_dims=(), collapsed_slice_dims=(0,), start_index_map=(0,))
    return lax.gather(x, idx[:, None], dnums, slice_sizes=(1,),
                      mode=lax.GatherScatterMode.PROMISE_IN_BOUNDS)


def _widen(u):
    lo = lax.bitcast_convert_type(u << 16, jnp.float32)
    hi = lax.bitcast_convert_type(u & jnp.int32(-65536), jnp.float32)
    return lo, hi


def _narrow(a, b):
    ua = lax.shift_right_logical(
        lax.bitcast_convert_type(a, jnp.int32) + jnp.int32(0x8000), 16)
    ub = (lax.bitcast_convert_type(b, jnp.int32) + jnp.int32(0x8000)) \
        & jnp.int32(-65536)
    return ub | (ua & jnp.int32(0xFFFF))


N_NODES_ = 10000
N_EDGES_ = 320000
D_ = 128

NC_ = 2
NS_ = 16
C_ = 64
NBUF_ = 4
NCHUNKS_ = N_EDGES_ // C_
CH_A_ = 190
CH_B_ = NCHUNKS_ // NS_ - CH_A_ - 1
_FAST_TOTAL_ = NS_ * CH_A_ + 8
CH_B_ = (NCHUNKS_ - _FAST_TOTAL_) // NS_
T_SRC_A_, T_DST_A_ = 8, 8
T_SRC_B_, T_DST_B_ = 8, 8
ROWS_T_ = N_NODES_ // NS_
RPK_ = 25


def _sc_kernel(hs, hd, ei, out, hpk_s, hpk_d,
               idx_s, idx_d, rows_s, rows_d, score, pin, pout, sh_s, sh_d,
               sem_i, sem_s, sem_d, sem_o):
    c = lax.axis_index("c")
    s = lax.axis_index("s")
    n = jnp.where(c == 0, CH_A_ + (s < 8), CH_B_).astype(jnp.int32)
    base = jnp.where(c == 0, s * CH_A_ + jnp.minimum(s, 8),
                     _FAST_TOTAL_ + s * CH_B_).astype(jnp.int32)
    t_src = jnp.where(c == 0, T_SRC_A_, T_SRC_B_).astype(jnp.int32)
    t_dst = jnp.where(c == 0, T_DST_A_, T_DST_B_).astype(jnp.int32)

    row0 = s * ROWS_T_

    def _pack(src_tab, dst_sh, dst_hbm):
        @pl.loop(0, ROWS_T_ // RPK_)
        def _blk(bk):
            r0 = row0 + bk * RPK_
            pltpu.sync_copy(src_tab.at[pl.ds(r0, RPK_)], pin)

            @pl.loop(0, RPK_)
            def _row(i):
                for q in range(4):
                    a = pin[i, 32 * q:32 * q + 16]
                    b = pin[i, 32 * q + 16:32 * q + 32]
                    pout[i, 16 * q:16 * (q + 1)] = _narrow(a, b)

            pltpu.sync_copy(pout, dst_sh.at[pl.ds(r0, RPK_)])
            pltpu.sync_copy(pout, dst_hbm.at[c, pl.ds(r0, RPK_)])

    _pack(hs, sh_s, hpk_s)
    _pack(hd, sh_d, hpk_d)
    plsc.subcore_barrier()

    lane = lax.iota(jnp.int32, 16)
    perms = [(lane + sh) & 15 for sh in (8, 4, 2, 1)]

    def _issue_idx(k):
        slot = k % NBUF_
        off = (base + k) * C_
        pltpu.async_copy(ei.at[0, pl.ds(off, C_)], idx_s.at[slot],
                         sem_i.at[slot])
        pltpu.async_copy(ei.at[1, pl.ds(off, C_)], idx_d.at[slot],
                         sem_i.at[slot])

    def _wait_idx(k):
        slot = k % NBUF_
        pltpu.make_async_copy(ei.at[0, pl.ds((base + k) * C_, C_)],
                              idx_s.at[slot], sem_i.at[slot]).wait()
        pltpu.make_async_copy(ei.at[1, pl.ds((base + k) * C_, C_)],
                              idx_d.at[slot], sem_i.at[slot]).wait()

    def _routes(k):
        sp_src = ((k * t_src) % 8) < t_src
        sp_dst = ((k * t_dst) % 8) < t_dst
        return sp_src, sp_dst

    def _issue_gather(k):
        slot = k % NBUF_
        sp_src, sp_dst = _routes(k)

        @pl.when(sp_src)
        def _():
            pltpu.async_copy(sh_s.at[idx_s.at[slot]], rows_s.at[slot],
                             sem_s.at[slot])

        @pl.when(jnp.logical_not(sp_src))
        def _():
            pltpu.async_copy(hpk_s.at[c].at[idx_s.at[slot]], rows_s.at[slot],
                             sem_s.at[slot])

        @pl.when(sp_dst)
        def _():
            pltpu.async_copy(sh_d.at[idx_d.at[slot]], rows_d.at[slot],
                             sem_d.at[slot])

        @pl.when(jnp.logical_not(sp_dst))
        def _():
            pltpu.async_copy(hpk_d.at[c].at[idx_d.at[slot]], rows_d.at[slot],
                             sem_d.at[slot])

    def _wait_gather(k):
        slot = k % NBUF_
        sp_src, sp_dst = _routes(k)

        @pl.when(sp_src)
        def _():
            pltpu.make_async_copy(sh_s.at[idx_s.at[slot]], rows_s.at[slot],
                                  sem_s.at[slot]).wait()

        @pl.when(jnp.logical_not(sp_src))
        def _():
            pltpu.make_async_copy(hpk_s.at[c].at[idx_s.at[slot]],
                                  rows_s.at[slot], sem_s.at[slot]).wait()

        @pl.when(sp_dst)
        def _():
            pltpu.make_async_copy(sh_d.at[idx_d.at[slot]], rows_d.at[slot],
                                  sem_d.at[slot]).wait()

        @pl.when(jnp.logical_not(sp_dst))
        def _():
            pltpu.make_async_copy(hpk_d.at[c].at[idx_d.at[slot]],
                                  rows_d.at[slot], sem_d.at[slot]).wait()

    def _issue_out(k):
        slot = k % NBUF_
        pltpu.async_copy(score.at[slot], out.at[pl.ds((base + k) * C_, C_)],
                         sem_o.at[slot])

    def _wait_out(k):
        slot = k % NBUF_
        pltpu.make_async_copy(score.at[slot],
                              out.at[pl.ds((base + k) * C_, C_)],
                              sem_o.at[slot]).wait()

    for j in range(NBUF_):
        _issue_idx(j)
    for j in range(NBUF_ - 1):
        _wait_idx(j)
        _issue_gather(j)

    @pl.loop(0, n)
    def _chunk(k):
        slot = k % NBUF_
        _wait_gather(k)

        @pl.when(k >= NBUF_)
        def _drain_out():
            _wait_out(k - NBUF_)

        @pl.loop(0, C_ // 16)
        def _group(g):
            @pl.loop(0, 16, init_carry=jnp.zeros((16,), jnp.float32))
            def _edge(u, sc_vec):
                e = g * 16 + u
                p = []
                for f in range(4):
                    a0, a1 = _widen(rows_s[slot, e, 16 * f:16 * (f + 1)])
                    b0, b1 = _widen(rows_d[slot, e, 16 * f:16 * (f + 1)])
                    p.append(a0 * b0)
                    p.append(a1 * b1)
                acc = ((p[0] + p[1]) + (p[2] + p[3])) + \
                      ((p[4] + p[5]) + (p[6] + p[7]))
                for pm in perms:
                    acc = acc + _lane_shuffle(acc, pm)
                return jnp.where(lane == u, acc, sc_vec)
            score[slot, pl.ds(g * 16, 16)] = _edge

        _issue_out(k)

        @pl.when(k + NBUF_ < n)
        def _refill_idx():
            _issue_idx(k + NBUF_)

        @pl.when(k + NBUF_ - 1 < n)
        def _refill_gather():
            _wait_idx(k + NBUF_ - 1)
            _issue_gather(k + NBUF_ - 1)

    for j in range(NBUF_):
        _wait_out(n - NBUF_ + j)


@jax.jit
def kernel(h_src, h_dst, edge_index):
    ei = edge_index.astype(jnp.int32)

    mesh = plsc.VectorSubcoreMesh(core_axis_name="c", subcore_axis_name="s")
    sck = functools.partial(
        pl.kernel,
        out_type=(
            jax.ShapeDtypeStruct((N_EDGES_,), jnp.float32),
            jax.ShapeDtypeStruct((NC_, N_NODES_, D_ // 2), jnp.int32),
            jax.ShapeDtypeStruct((NC_, N_NODES_, D_ // 2), jnp.int32),
        ),
        mesh=mesh,
        compiler_params=pltpu.CompilerParams(use_tc_tiling_on_sc=False),
        scratch_types=[
            pltpu.VMEM((NBUF_, C_), jnp.int32),
            pltpu.VMEM((NBUF_, C_), jnp.int32),
            pltpu.VMEM((NBUF_, C_, D_ // 2), jnp.int32),
            pltpu.VMEM((NBUF_, C_, D_ // 2), jnp.int32),
            pltpu.VMEM((NBUF_, C_), jnp.float32),
            pltpu.VMEM((RPK_, D_), jnp.float32),
            pltpu.VMEM((RPK_, D_ // 2), jnp.int32),
            pltpu.VMEM_SHARED((N_NODES_, D_ // 2), jnp.int32),
            pltpu.VMEM_SHARED((N_NODES_, D_ // 2), jnp.int32),
            pltpu.SemaphoreType.DMA((NBUF_,)),
            pltpu.SemaphoreType.DMA((NBUF_,)),
            pltpu.SemaphoreType.DMA((NBUF_,)),
            pltpu.SemaphoreType.DMA((NBUF_,)),
        ],
    )(_sc_kernel)
    out, _, _ = sck(h_src, h_dst, ei)
    return out.reshape(N_EDGES_, 1)
